# baseline (speedup 1.0000x reference)
import functools

import jax
import jax.numpy as jnp
from jax import lax
from jax.experimental import pallas as pl
from jax.experimental.pallas import tpu as pltpu
from jax.experimental.pallas import tpu_sc as plsc

N_NODES = 10000
N_PAD = 10240
D = 128
NC = 2
NS = 16
NW = NC * NS
LANES = 16
CHUNK = 128
ROWS_PER_SC_TILE = N_PAD // NS
ZCHUNK = 128
SUPER = 8


def _sc_kernel_body(x_hbm, src_hbm, dst_hbm, norm_hbm, part_hbm,
                    src_v, dst_v, norm_v, rows_a, rows_b, acc, sem_a, sem_b):
    cid = lax.axis_index("c")
    sid = lax.axis_index("s")
    wid = cid * NS + sid
    n_chunks = src_hbm.shape[0] // NW
    n_super = n_chunks // SUPER

    def zrow(e, _):
        for t in range(D // LANES):
            rows_a[e, pl.ds(t * LANES, LANES)] = jnp.zeros((LANES,), jnp.float32)
        return 0
    lax.fori_loop(0, ZCHUNK, zrow, 0)
    row0 = sid * ROWS_PER_SC_TILE
    for q in range(ROWS_PER_SC_TILE // ZCHUNK):
        pltpu.sync_copy(rows_a, acc.at[pl.ds(row0 + q * ZCHUNK, ZCHUNK)])
    plsc.subcore_barrier()

    c0 = wid * n_chunks

    def gather(j, rows, sem):
        pltpu.async_copy(x_hbm.at[src_v.at[j]], rows, sem)

    def wait_rows(rows, sem):
        pltpu.make_async_copy(x_hbm.at[pl.ds(0, CHUNK)], rows, sem).wait()

    def consume(j, rows):
        def scale(g, _):
            nv16 = norm_v[pl.ds(j * CHUNK + g * LANES, LANES)]
            for l in range(LANES):
                nv = jnp.full((LANES,), nv16[l])
                e = g * LANES + l
                for t in range(D // LANES):
                    sl = pl.ds(t * LANES, LANES)
                    rows[e, sl] = rows[e, sl] * nv
            return 0
        lax.fori_loop(0, CHUNK // LANES, scale, 0)
        pltpu.sync_copy(rows, acc.at[dst_v.at[j]], add=True)

    def super_body(s, _):
        pltpu.sync_copy(src_hbm.at[pl.ds(c0 + s * SUPER, SUPER)], src_v)
        pltpu.sync_copy(dst_hbm.at[pl.ds(c0 + s * SUPER, SUPER)], dst_v)
        pltpu.sync_copy(
            norm_hbm.at[pl.ds((c0 + s * SUPER) * CHUNK, SUPER * CHUNK)], norm_v)

        gather(0, rows_a, sem_a)

        def pair(st, _):
            j0 = st * 2
            wait_rows(rows_a, sem_a)
            gather(j0 + 1, rows_b, sem_b)
            consume(j0, rows_a)
            wait_rows(rows_b, sem_b)

            @pl.when(j0 + 2 < SUPER)
            def _():
                gather(j0 + 2, rows_a, sem_a)
            consume(j0 + 1, rows_b)
            return 0
        lax.fori_loop(0, SUPER // 2, pair, 0)
        return 0
    lax.fori_loop(0, n_super, super_body, 0)
    plsc.subcore_barrier()

    for q in range(ROWS_PER_SC_TILE // ZCHUNK):
        r = row0 + q * ZCHUNK
        pltpu.sync_copy(acc.at[pl.ds(r, ZCHUNK)], part_hbm.at[cid, pl.ds(r, ZCHUNK)])


def _make_sc_call(n_chunk_rows):
    mesh = plsc.VectorSubcoreMesh(core_axis_name="c", subcore_axis_name="s")
    return pl.kernel(
        _sc_kernel_body,
        mesh=mesh,
        compiler_params=pltpu.CompilerParams(use_tc_tiling_on_sc=False),
        out_type=jax.ShapeDtypeStruct((NC, N_PAD, D), jnp.float32),
        scratch_types=[
            pltpu.VMEM((SUPER, CHUNK), jnp.int32),
            pltpu.VMEM((SUPER, CHUNK), jnp.int32),
            pltpu.VMEM((SUPER * CHUNK,), jnp.float32),
            pltpu.VMEM((CHUNK, D), jnp.float32),
            pltpu.VMEM((CHUNK, D), jnp.float32),
            pltpu.VMEM_SHARED((N_PAD, D), jnp.float32),
            pltpu.SemaphoreType.DMA,
            pltpu.SemaphoreType.DMA,
        ],
    )


def _add_body(a_ref, b_ref, o_ref):
    o_ref[...] = a_ref[...] + b_ref[...]


_combine = pl.pallas_call(
    _add_body,
    grid=(10,),
    in_specs=[pl.BlockSpec((N_PAD // 10, D), lambda i: (i, 0))] * 2,
    out_specs=pl.BlockSpec((N_PAD // 10, D), lambda i: (i, 0)),
    out_shape=jax.ShapeDtypeStruct((N_PAD, D), jnp.float32),
)


@jax.jit
def kernel(x, edge_index, edge_norm):
    src = edge_index[0].astype(jnp.int32)
    dst = edge_index[1].astype(jnp.int32)
    norm = edge_norm.reshape(-1).astype(jnp.float32)
    e = src.shape[0]
    per_worker_chunks = -(-e // (NW * CHUNK))
    per_worker_chunks = -(-per_worker_chunks // 8) * 8
    e_pad = per_worker_chunks * NW * CHUNK
    pad = e_pad - e
    if pad:
        fill = (jnp.arange(pad, dtype=jnp.int32) * 37) % N_NODES
        src = jnp.concatenate([src, fill])
        dst = jnp.concatenate([dst, fill])
        norm = jnp.concatenate([norm, jnp.zeros((pad,), jnp.float32)])
    n_chunk_rows = e_pad // CHUNK
    src2 = src.reshape(n_chunk_rows, CHUNK)
    dst2 = dst.reshape(n_chunk_rows, CHUNK)
    part = _make_sc_call(n_chunk_rows)(x, src2, dst2, norm)
    return _combine(part[0], part[1])[:N_NODES]


# speedup vs baseline: 1.0527x; 1.0527x over previous
import functools

import jax
import jax.numpy as jnp
from jax import lax
from jax.experimental import pallas as pl
from jax.experimental.pallas import tpu as pltpu
from jax.experimental.pallas import tpu_sc as plsc

N_NODES = 10000
N_PAD = 10240
D = 128
NC = 2
NS = 16
NW = NC * NS
LANES = 16
CHUNK = 128
ROWS_PER_SC_TILE = N_PAD // NS
ZCHUNK = 128
SUPER = 8


def _sc_kernel_body(x_hbm, src_hbm, dst_hbm, norm_hbm, part_hbm,
                    src_v, dst_v, norm_v, rows_a, rows_b, acc, sem_a, sem_b):
    cid = lax.axis_index("c")
    sid = lax.axis_index("s")
    wid = cid * NS + sid
    n_chunks = src_hbm.shape[0] // NW
    n_super = n_chunks // SUPER

    def zrow(e, _):
        for t in range(D // LANES):
            rows_a[e, pl.ds(t * LANES, LANES)] = jnp.zeros((LANES,), jnp.float32)
        return 0
    lax.fori_loop(0, ZCHUNK, zrow, 0)
    row0 = sid * ROWS_PER_SC_TILE
    for q in range(ROWS_PER_SC_TILE // ZCHUNK):
        pltpu.sync_copy(rows_a, acc.at[pl.ds(row0 + q * ZCHUNK, ZCHUNK)])
    plsc.subcore_barrier()

    c0 = wid * n_chunks

    def gather(j, rows, sem):
        pltpu.async_copy(x_hbm.at[src_v.at[j]], rows, sem)

    def wait_rows(rows, sem):
        pltpu.make_async_copy(x_hbm.at[pl.ds(0, CHUNK)], rows, sem).wait()

    def consume(j, rows):
        def scale(g, _):
            nv16 = norm_v[pl.ds(j * CHUNK + g * LANES, LANES)]
            for l in range(LANES):
                nv = jnp.full((LANES,), nv16[l])
                e = g * LANES + l
                for t in range(D // LANES):
                    sl = pl.ds(t * LANES, LANES)
                    rows[e, sl] = rows[e, sl] * nv
            return 0
        lax.fori_loop(0, CHUNK // LANES, scale, 0)
        pltpu.sync_copy(rows, acc.at[dst_v.at[j]], add=True)

    def super_body(s, _):
        pltpu.sync_copy(src_hbm.at[pl.ds(c0 + s * SUPER, SUPER)], src_v)
        pltpu.sync_copy(dst_hbm.at[pl.ds(c0 + s * SUPER, SUPER)], dst_v)
        pltpu.sync_copy(
            norm_hbm.at[pl.ds((c0 + s * SUPER) * CHUNK, SUPER * CHUNK)], norm_v)

        gather(0, rows_a, sem_a)

        def pair(st, _):
            j0 = st * 2
            wait_rows(rows_a, sem_a)
            gather(j0 + 1, rows_b, sem_b)
            consume(j0, rows_a)
            wait_rows(rows_b, sem_b)

            @pl.when(j0 + 2 < SUPER)
            def _():
                gather(j0 + 2, rows_a, sem_a)
            consume(j0 + 1, rows_b)
            return 0
        lax.fori_loop(0, SUPER // 2, pair, 0)
        return 0
    lax.fori_loop(0, n_super, super_body, 0)
    plsc.subcore_barrier()

    for q in range(ROWS_PER_SC_TILE // ZCHUNK):
        r = row0 + q * ZCHUNK
        pltpu.sync_copy(acc.at[pl.ds(r, ZCHUNK)], part_hbm.at[cid, pl.ds(r, ZCHUNK)])


def _make_sc_call(n_chunk_rows):
    mesh = plsc.VectorSubcoreMesh(core_axis_name="c", subcore_axis_name="s")
    return pl.kernel(
        _sc_kernel_body,
        mesh=mesh,
        out_type=jax.ShapeDtypeStruct((NC, N_PAD, D), jnp.float32),
        scratch_types=[
            pltpu.VMEM((SUPER, CHUNK), jnp.int32),
            pltpu.VMEM((SUPER, CHUNK), jnp.int32),
            pltpu.VMEM((SUPER * CHUNK,), jnp.float32),
            pltpu.VMEM((CHUNK, D), jnp.float32),
            pltpu.VMEM((CHUNK, D), jnp.float32),
            pltpu.VMEM_SHARED((N_PAD, D), jnp.float32),
            pltpu.SemaphoreType.DMA,
            pltpu.SemaphoreType.DMA,
        ],
    )


def _add_body(a_ref, b_ref, o_ref):
    o_ref[...] = a_ref[0] + b_ref[0]


_combine = pl.pallas_call(
    _add_body,
    grid=(10,),
    in_specs=[pl.BlockSpec((1, N_NODES // 10, D), lambda i: (0, i, 0)),
              pl.BlockSpec((1, N_NODES // 10, D), lambda i: (1, i, 0))],
    out_specs=pl.BlockSpec((N_NODES // 10, D), lambda i: (i, 0)),
    out_shape=jax.ShapeDtypeStruct((N_NODES, D), jnp.float32),
)


@jax.jit
def kernel(x, edge_index, edge_norm):
    src = edge_index[0].astype(jnp.int32)
    dst = edge_index[1].astype(jnp.int32)
    norm = edge_norm.reshape(-1).astype(jnp.float32)
    e = src.shape[0]
    per_worker_chunks = -(-e // (NW * CHUNK))
    per_worker_chunks = -(-per_worker_chunks // 8) * 8
    e_pad = per_worker_chunks * NW * CHUNK
    pad = e_pad - e
    if pad:
        fill = (jnp.arange(pad, dtype=jnp.int32) * 37) % N_NODES
        src = jnp.concatenate([src, fill])
        dst = jnp.concatenate([dst, fill])
        norm = jnp.concatenate([norm, jnp.zeros((pad,), jnp.float32)])
    n_chunk_rows = e_pad // CHUNK
    src2 = src.reshape(n_chunk_rows, CHUNK)
    dst2 = dst.reshape(n_chunk_rows, CHUNK)
    part = _make_sc_call(n_chunk_rows)(x, src2, dst2, norm)
    return _combine(part, part)


# speedup vs baseline: 1.1441x; 1.0868x over previous
import functools

import jax
import jax.numpy as jnp
from jax import lax
from jax.experimental import pallas as pl
from jax.experimental.pallas import tpu as pltpu
from jax.experimental.pallas import tpu_sc as plsc

N_NODES = 10000
N_PAD = 10240
D = 128
NC = 2
NS = 16
NW = NC * NS
LANES = 16
CHUNK = 128
ROWS_PER_SC_TILE = N_PAD // NS
ZCHUNK = 128


def _pick_super(n_full):
    for s in (26, 20, 16, 12, 10, 8, 6, 4, 2):
        if n_full % s == 0:
            return s
    return 1


def _sc_kernel_body(super_, x_hbm, src_hbm, dst_hbm, norm_hbm, part_hbm,
                    src_v, dst_v, norm_v, srow, drow, nrow,
                    rows_a, rows_b, acc, sem_a, sem_b):
    cid = lax.axis_index("c")
    sid = lax.axis_index("s")
    wid = cid * NS + sid
    epw = src_hbm.shape[0] // NW
    n_full = epw // CHUNK
    tail = epw - n_full * CHUNK
    n_super = n_full // super_
    e0 = wid * epw

    def zrow(e, _):
        for t in range(D // LANES):
            rows_a[e, pl.ds(t * LANES, LANES)] = jnp.zeros((LANES,), jnp.float32)
        return 0
    lax.fori_loop(0, ZCHUNK, zrow, 0)
    row0 = sid * ROWS_PER_SC_TILE
    for q in range(ROWS_PER_SC_TILE // ZCHUNK):
        pltpu.sync_copy(rows_a, acc.at[pl.ds(row0 + q * ZCHUNK, ZCHUNK)])
    plsc.subcore_barrier()

    def gather(idx_ref, rows, sem):
        pltpu.async_copy(x_hbm.at[idx_ref], rows, sem)

    def wait_rows(rows, sem):
        pltpu.make_async_copy(x_hbm.at[pl.ds(0, CHUNK)], rows, sem).wait()

    def scale_rows(rows, get_nv16):
        def grp(g, _):
            nv16 = get_nv16(g)
            for l in range(LANES):
                nv = jnp.full((LANES,), nv16[l])
                e = g * LANES + l
                for t in range(D // LANES):
                    sl = pl.ds(t * LANES, LANES)
                    rows[e, sl] = rows[e, sl] * nv
            return 0
        lax.fori_loop(0, CHUNK // LANES, grp, 0)

    def consume(j, rows):
        scale_rows(rows, lambda g: norm_v[pl.ds(j * CHUNK + g * LANES, LANES)])
        for t in range(D // LANES):
            drow[0, pl.ds(t * LANES, LANES)] = dst_v[pl.ds(j * CHUNK + t * LANES,
                                                           LANES)]
        pltpu.sync_copy(rows, acc.at[drow.at[0]], add=True)

    def super_body(s, _):
        base = e0 + s * (super_ * CHUNK)
        pltpu.sync_copy(src_hbm.at[pl.ds(base, super_ * CHUNK)], src_v)
        pltpu.sync_copy(dst_hbm.at[pl.ds(base, super_ * CHUNK)], dst_v)
        pltpu.sync_copy(norm_hbm.at[pl.ds(base, super_ * CHUNK)], norm_v)

        gather(src_v.at[pl.ds(0, CHUNK)], rows_a, sem_a)

        def pair(st, _):
            j0 = st * 2
            wait_rows(rows_a, sem_a)
            gather(src_v.at[pl.ds((j0 + 1) * CHUNK, CHUNK)], rows_b, sem_b)
            consume(j0, rows_a)
            wait_rows(rows_b, sem_b)

            @pl.when(j0 + 2 < super_)
            def _():
                gather(src_v.at[pl.ds((j0 + 2) * CHUNK, CHUNK)], rows_a, sem_a)
            consume(j0 + 1, rows_b)
            return 0
        if super_ > 1:
            lax.fori_loop(0, super_ // 2, pair, 0)
        else:
            wait_rows(rows_a, sem_a)
            consume(0, rows_a)
        return 0
    lax.fori_loop(0, n_super, super_body, 0)

    if tail:
        iota = lax.iota(jnp.int32, LANES)
        for t in range(CHUNK // LANES):
            spread = iota + t * LANES
            srow[0, pl.ds(t * LANES, LANES)] = spread
            drow[0, pl.ds(t * LANES, LANES)] = spread
            nrow[0, pl.ds(t * LANES, LANES)] = jnp.zeros((LANES,), jnp.float32)
        tb = e0 + n_full * CHUNK
        pltpu.sync_copy(src_hbm.at[pl.ds(tb, tail)], srow.at[0, pl.ds(0, tail)])
        pltpu.sync_copy(dst_hbm.at[pl.ds(tb, tail)], drow.at[0, pl.ds(0, tail)])
        pltpu.sync_copy(norm_hbm.at[pl.ds(tb, tail)], nrow.at[0, pl.ds(0, tail)])
        gather(srow.at[0], rows_a, sem_a)
        wait_rows(rows_a, sem_a)
        scale_rows(rows_a, lambda g: nrow[0, pl.ds(g * LANES, LANES)])
        pltpu.sync_copy(rows_a, acc.at[drow.at[0]], add=True)

    plsc.subcore_barrier()

    for q in range(ROWS_PER_SC_TILE // ZCHUNK):
        r = row0 + q * ZCHUNK
        pltpu.sync_copy(acc.at[pl.ds(r, ZCHUNK)], part_hbm.at[cid, pl.ds(r, ZCHUNK)])


def _make_sc_call(e_pad):
    n_full = (e_pad // NW) // CHUNK
    super_ = _pick_super(n_full)
    mesh = plsc.VectorSubcoreMesh(core_axis_name="c", subcore_axis_name="s")
    return pl.kernel(
        functools.partial(_sc_kernel_body, super_),
        mesh=mesh,
        out_type=jax.ShapeDtypeStruct((NC, N_PAD, D), jnp.float32),
        scratch_types=[
            pltpu.VMEM((super_ * CHUNK,), jnp.int32),
            pltpu.VMEM((super_ * CHUNK,), jnp.int32),
            pltpu.VMEM((super_ * CHUNK,), jnp.float32),
            pltpu.VMEM((1, CHUNK), jnp.int32),
            pltpu.VMEM((1, CHUNK), jnp.int32),
            pltpu.VMEM((1, CHUNK), jnp.float32),
            pltpu.VMEM((CHUNK, D), jnp.float32),
            pltpu.VMEM((CHUNK, D), jnp.float32),
            pltpu.VMEM_SHARED((N_PAD, D), jnp.float32),
            pltpu.SemaphoreType.DMA,
            pltpu.SemaphoreType.DMA,
        ],
    )


def _add_body(a_ref, b_ref, o_ref):
    o_ref[...] = a_ref[0] + b_ref[0]


_combine = pl.pallas_call(
    _add_body,
    grid=(10,),
    in_specs=[pl.BlockSpec((1, N_NODES // 10, D), lambda i: (0, i, 0)),
              pl.BlockSpec((1, N_NODES // 10, D), lambda i: (1, i, 0))],
    out_specs=pl.BlockSpec((N_NODES // 10, D), lambda i: (i, 0)),
    out_shape=jax.ShapeDtypeStruct((N_NODES, D), jnp.float32),
)


@jax.jit
def kernel(x, edge_index, edge_norm):
    src = edge_index[0].astype(jnp.int32)
    dst = edge_index[1].astype(jnp.int32)
    norm = edge_norm.reshape(-1).astype(jnp.float32)
    e = src.shape[0]
    epw = (((e + NW - 1) // NW + 7) // 8) * 8
    e_pad = epw * NW
    pad = e_pad - e
    if pad:
        fill = (jnp.arange(pad, dtype=jnp.int32) * 37) % N_NODES
        src = jnp.concatenate([src, fill])
        dst = jnp.concatenate([dst, fill])
        norm = jnp.concatenate([norm, jnp.zeros((pad,), jnp.float32)])
    part = _make_sc_call(e_pad)(x, src, dst, norm)
    return _combine(part, part)
